# baseline (speedup 1.0000x reference)
import functools

import jax
import jax.numpy as jnp
from jax import lax
from jax.experimental import pallas as pl
from jax.experimental.pallas import tpu as pltpu
from jax.experimental.pallas import tpu_sc as plsc

N_TOKENS = 32768
N_EXPERTS = 64
NC = 2
NS = 16
L = 16
NW = NC * NS
ROWS_PER_W = N_TOKENS // NW
OUT_WORDS = N_TOKENS * N_EXPERTS
WORDS_PER_W = ROWS_PER_W * N_EXPERTS

ZWORDS_TILE = 1024
ZWORDS = ZWORDS_TILE * NS
NZDMA = WORDS_PER_W // ZWORDS

IDX_MINOR = 128
IDX_ROWS = ROWS_PER_W // IDX_MINOR


TC_BLOCK_ROWS = 2048
TC_GRID = N_TOKENS // TC_BLOCK_ROWS


def _argmax_body(x_ref, idx_ref):
    xb = x_ref[...]
    mx = jnp.max(xb, axis=1, keepdims=True)
    cols = lax.broadcasted_iota(jnp.int32, xb.shape, 1)
    cand = jnp.where(xb == mx, cols, N_EXPERTS)
    idx = jnp.min(cand, axis=1)
    idx_ref[...] = idx.reshape(idx_ref.shape)


_tc_argmax = pl.pallas_call(
    _argmax_body,
    grid=(TC_GRID,),
    in_specs=[pl.BlockSpec((TC_BLOCK_ROWS, N_EXPERTS), lambda i: (i, 0))],
    out_specs=pl.BlockSpec((TC_BLOCK_ROWS // 128, 128), lambda i: (i, 0)),
    out_shape=jax.ShapeDtypeStruct((N_TOKENS // 128, 128), jnp.int32),
)


@functools.partial(
    pl.kernel,
    out_type=jax.ShapeDtypeStruct((OUT_WORDS,), jnp.float32),
    mesh=plsc.VectorSubcoreMesh(core_axis_name="c", subcore_axis_name="s"),
    scratch_types=[
        pltpu.VMEM((ZWORDS_TILE,), jnp.float32),
        pltpu.VMEM((ROWS_PER_W,), jnp.int32),
        pltpu.VMEM((IDX_ROWS, IDX_MINOR), jnp.int32),
        pltpu.VMEM((IDX_MINOR,), jnp.float32),
        pltpu.VMEM_SHARED((ZWORDS,), jnp.float32),
        pltpu.SemaphoreType.DMA,
        pltpu.SemaphoreType.DMA,
    ],
)
def _sc_onehot(idx_hbm, out_hbm, zt_v, idx_v, off_v, ones_v, sh_zero, zsem,
               ssem):
    c = lax.axis_index("c")
    s = lax.axis_index("s")
    wid = s * NC + c
    row_base = wid * ROWS_PER_W
    word_base = row_base * N_EXPERTS
    lane = lax.iota(jnp.int32, L)
    zeros = jnp.zeros((L,), jnp.float32)
    ones = jnp.full((L,), 1.0, jnp.float32)

    @pl.loop(0, ZWORDS_TILE // L)
    def _fill(i):
        zt_v[pl.ds(i * L, L)] = zeros

    @pl.loop(0, IDX_MINOR // L)
    def _fill1(i):
        ones_v[pl.ds(i * L, L)] = ones

    pltpu.sync_copy(idx_hbm.at[pl.ds(row_base, ROWS_PER_W)], idx_v)
    pltpu.sync_copy(zt_v, sh_zero.at[pl.ds(s * ZWORDS_TILE, ZWORDS_TILE)])
    plsc.subcore_barrier()

    zcopies = []
    for d in range(NZDMA):
        zcopies.append(pltpu.async_copy(
            sh_zero, out_hbm.at[pl.ds(word_base + d * ZWORDS, ZWORDS)], zsem))

    @pl.loop(0, ROWS_PER_W // L)
    def _offs(j):
        r = j * L + lane
        off = (row_base + r) * N_EXPERTS + idx_v[pl.ds(j * L, L)]
        off_v[j // (IDX_MINOR // L), pl.ds((j % (IDX_MINOR // L)) * L, L)] = off

    for cp in zcopies:
        cp.wait()

    scopies = []
    for j in range(IDX_ROWS):
        scopies.append(pltpu.async_copy(ones_v, out_hbm.at[off_v.at[j]], ssem))
    for cp in scopies:
        cp.wait()


def kernel(x):
    idx = _tc_argmax(x)
    out = _sc_onehot(idx.reshape(N_TOKENS))
    return out.reshape(N_TOKENS, N_EXPERTS)
